# baseline (speedup 1.0000x reference)
import functools
import math

import jax
import jax.numpy as jnp
from jax import lax
from jax.experimental import pallas as pl
from jax.experimental.pallas import tpu as pltpu
from jax.experimental.pallas import tpu_sc as plsc

D = 256
VOCAB = 8192
SCALE = math.sqrt(D)
S2_BITS = 13
S2_MASK = (1 << S2_BITS) - 1


_ROWS_PER_BLOCK = 1024
_N_BLOCKS = VOCAB // _ROWS_PER_BLOCK


def _pack_bf16(x):
    lo = lax.bitcast_convert_type(
        x[:, : D // 2].astype(jnp.bfloat16), jnp.uint16
    ).astype(jnp.uint32)
    hi = lax.bitcast_convert_type(
        x[:, D // 2 :].astype(jnp.bfloat16), jnp.uint16
    ).astype(jnp.uint32)
    return (lo | (hi << 16)).astype(jnp.int32)


def _transform_body(e1_ref, e2_ref, w1_ref, w2_ref, b_ref, t1_ref, t2_ref):
    x1 = (
        jnp.dot(e1_ref[...], w1_ref[...], preferred_element_type=jnp.float32)
        * SCALE
        + b_ref[...]
    )
    x2 = (
        jnp.dot(e2_ref[...], w2_ref[...], preferred_element_type=jnp.float32)
        * SCALE
    )
    t1_ref[...] = _pack_bf16(x1)
    t2_ref[...] = _pack_bf16(x2)


def _transform_tables(emb_s1, emb_s2, w1, w2, b2d):
    blk = _ROWS_PER_BLOCK
    return pl.pallas_call(
        _transform_body,
        grid=(_N_BLOCKS,),
        in_specs=[
            pl.BlockSpec((blk, D), lambda i: (i, 0)),
            pl.BlockSpec((blk, D), lambda i: (i, 0)),
            pl.BlockSpec((D, D), lambda i: (0, 0)),
            pl.BlockSpec((D, D), lambda i: (0, 0)),
            pl.BlockSpec((1, D), lambda i: (0, 0)),
        ],
        out_specs=[
            pl.BlockSpec((blk, D // 2), lambda i: (i, 0)),
            pl.BlockSpec((blk, D // 2), lambda i: (i, 0)),
        ],
        out_shape=[
            jax.ShapeDtypeStruct((VOCAB, D // 2), jnp.int32),
            jax.ShapeDtypeStruct((VOCAB, D // 2), jnp.int32),
        ],
    )(emb_s1, emb_s2, w1, w2, b2d)


_NTOK = 32768
_NW = 32
_TPW = _NTOK // _NW
_C = 64
_NCHUNK = _TPW // _C
_L = 16


def _gather_add_body(tok_hbm, t1_hbm, t2_hbm, out_hbm,
                     tok_v, idx1_v, idx2_v,
                     bufa0_v, bufa1_v, bufb0_v, bufb1_v, bufo0_v, bufo1_v,
                     sem_g, sem_a, sem_o):
    wid = lax.axis_index("s") * 2 + lax.axis_index("c")
    base = wid * _TPW
    bufa = [bufa0_v, bufa1_v]
    bufb = [bufb0_v, bufb1_v]
    bufo = [bufo0_v, bufo1_v]

    pltpu.sync_copy(tok_hbm.at[pl.ds(base, _TPW)], tok_v)

    def split_body(j, _):
        sl = pl.ds(j * _L, _L)
        t = tok_v[sl]
        idx1_v[sl] = lax.shift_right_logical(t, S2_BITS)
        idx2_v[sl] = lax.bitwise_and(t, S2_MASK)
        return 0

    lax.fori_loop(0, _TPW // _L, split_body, 0)

    def fire_g(c, p):
        i1 = idx1_v.at[pl.ds(c * _C, _C)]
        i2 = idx2_v.at[pl.ds(c * _C, _C)]
        return (pltpu.async_copy(t1_hbm.at[i1], bufa[p], sem_g[p]),
                pltpu.async_copy(t2_hbm.at[i2], bufb[p], sem_a[p]))

    def add_chunk(p):
        a = bufa[p]
        b = bufb[p]
        o = bufo[p]

        def add_row(r, _):
            for k in range(D // 2 // _L):
                sl = pl.ds(k * _L, _L)
                wa = plsc.bitcast(a[r, sl], jnp.bfloat16)
                wb = plsc.bitcast(b[r, sl], jnp.bfloat16)
                alo, ahi = plsc.unpack(wa, format=plsc.PackFormat.INTERLEAVED)
                blo, bhi = plsc.unpack(wb, format=plsc.PackFormat.INTERLEAVED)
                o[r, sl] = alo + blo
                o[r, pl.ds(D // 2 + k * _L, _L)] = ahi + bhi
            return 0

        lax.fori_loop(0, _C, add_row, 0)

    pending_o = [None, None]
    g = [None, None]
    g[0] = fire_g(0, 0)

    for c in range(_NCHUNK):
        p = c & 1
        q = 1 - p
        g[p][0].wait()
        g[p][1].wait()
        if c + 1 < _NCHUNK:
            if pending_o[q] is not None:
                pending_o[q].wait()
            g[q] = fire_g(c + 1, q)
        add_chunk(p)
        pending_o[p] = pltpu.async_copy(
            bufo[p], out_hbm.at[pl.ds(base + c * _C, _C)], sem_o[p])

    pending_o[0].wait()
    pending_o[1].wait()


def _gather_add(tok, t1, t2):
    mesh = plsc.VectorSubcoreMesh(core_axis_name="c", subcore_axis_name="s")
    fn = functools.partial(
        pl.kernel,
        mesh=mesh,
        out_type=jax.ShapeDtypeStruct((_NTOK, D), jnp.float32),
        scratch_types=[
            pltpu.VMEM((_TPW,), jnp.int32),
            pltpu.VMEM((_TPW,), jnp.int32),
            pltpu.VMEM((_TPW,), jnp.int32),
            pltpu.VMEM((_C, D // 2), jnp.int32),
            pltpu.VMEM((_C, D // 2), jnp.int32),
            pltpu.VMEM((_C, D // 2), jnp.int32),
            pltpu.VMEM((_C, D // 2), jnp.int32),
            pltpu.VMEM((_C, D), jnp.float32),
            pltpu.VMEM((_C, D), jnp.float32),
            [pltpu.SemaphoreType.DMA, pltpu.SemaphoreType.DMA],
            [pltpu.SemaphoreType.DMA, pltpu.SemaphoreType.DMA],
            [pltpu.SemaphoreType.DMA, pltpu.SemaphoreType.DMA],
        ],
        compiler_params=pltpu.CompilerParams(needs_layout_passes=False),
    )(_gather_add_body)
    return fn(tok, t1, t2)


def kernel(token_ids, emb_s1, emb_s2, W_fuse, b_fuse):
    w1 = W_fuse[:D]
    w2 = W_fuse[D:]
    b2d = b_fuse.reshape(1, D)
    t1, t2 = _transform_tables(emb_s1, emb_s2, w1, w2, b2d)
    tok = token_ids.reshape(-1)
    out = _gather_add(tok, t1, t2)
    return out.reshape(token_ids.shape + (D,))


# speedup vs baseline: 1.2330x; 1.2330x over previous
import functools
import math

import jax
import jax.numpy as jnp
from jax import lax
from jax.experimental import pallas as pl
from jax.experimental.pallas import tpu as pltpu
from jax.experimental.pallas import tpu_sc as plsc

D = 256
VOCAB = 8192
SCALE = math.sqrt(D)
S2_BITS = 13
S2_MASK = (1 << S2_BITS) - 1


_ROWS_PER_BLOCK = 1024
_N_BLOCKS = VOCAB // _ROWS_PER_BLOCK


def _transform_body(e1_ref, e2_ref, w1_ref, w2_ref, b_ref, t1_ref, t2_ref):
    t1_ref[...] = (
        jnp.dot(e1_ref[...], w1_ref[...], preferred_element_type=jnp.float32)
        * SCALE
        + b_ref[...]
    )
    t2_ref[...] = (
        jnp.dot(e2_ref[...], w2_ref[...], preferred_element_type=jnp.float32)
        * SCALE
    )


def _transform_tables(emb_s1, emb_s2, w1, w2, b2d):
    blk = _ROWS_PER_BLOCK
    return pl.pallas_call(
        _transform_body,
        grid=(_N_BLOCKS,),
        in_specs=[
            pl.BlockSpec((blk, D), lambda i: (i, 0)),
            pl.BlockSpec((blk, D), lambda i: (i, 0)),
            pl.BlockSpec((D, D), lambda i: (0, 0)),
            pl.BlockSpec((D, D), lambda i: (0, 0)),
            pl.BlockSpec((1, D), lambda i: (0, 0)),
        ],
        out_specs=[
            pl.BlockSpec((blk, D), lambda i: (i, 0)),
            pl.BlockSpec((blk, D), lambda i: (i, 0)),
        ],
        out_shape=[
            jax.ShapeDtypeStruct((VOCAB, D), jnp.float32),
            jax.ShapeDtypeStruct((VOCAB, D), jnp.float32),
        ],
    )(emb_s1, emb_s2, w1, w2, b2d)


_NTOK = 32768
_NW = 32
_TPW = _NTOK // _NW
_C = 64
_NCHUNK = _TPW // _C
_L = 16


def _gather_add_body(tok_hbm, t1_hbm, t2_hbm, out_hbm,
                     tok_v, idx1_v, idx2_v,
                     bufa0_v, bufa1_v, bufb0_v, bufb1_v,
                     sem_g, sem_a, sem_o):
    wid = lax.axis_index("s") * 2 + lax.axis_index("c")
    base = wid * _TPW
    bufa = [bufa0_v, bufa1_v]
    bufb = [bufb0_v, bufb1_v]

    pltpu.sync_copy(tok_hbm.at[pl.ds(base, _TPW)], tok_v)

    @plsc.parallel_loop(0, _TPW // _L, 1, unroll=4)
    def split_body(j):
        sl = pl.ds(j * _L, _L)
        t = tok_v[sl]
        idx1_v[sl] = lax.shift_right_logical(t, S2_BITS)
        idx2_v[sl] = lax.bitwise_and(t, S2_MASK)

    def fire_g(c, p):
        i1 = idx1_v.at[pl.ds(c * _C, _C)]
        i2 = idx2_v.at[pl.ds(c * _C, _C)]
        return (pltpu.async_copy(t1_hbm.at[i1], bufa[p], sem_g[p]),
                pltpu.async_copy(t2_hbm.at[i2], bufb[p], sem_a[p]))

    def add_chunk(p):
        a = bufa[p]
        b = bufb[p]

        @plsc.parallel_loop(0, _C, 1, unroll=4)
        def add_row(r):
            for k in range(D // _L):
                sl = pl.ds(k * _L, _L)
                plsc.addupdate(a.at[r, sl], b[r, sl])

    pending_o = [None, None]
    g = [None, None]
    g[0] = fire_g(0, 0)

    for c in range(_NCHUNK):
        p = c & 1
        q = 1 - p
        g[p][0].wait()
        g[p][1].wait()
        if c + 1 < _NCHUNK:
            if pending_o[q] is not None:
                pending_o[q].wait()
            g[q] = fire_g(c + 1, q)
        add_chunk(p)
        pending_o[p] = pltpu.async_copy(
            bufa[p], out_hbm.at[pl.ds(base + c * _C, _C)], sem_o[p])

    pending_o[0].wait()
    pending_o[1].wait()


def _gather_add(tok, t1, t2):
    mesh = plsc.VectorSubcoreMesh(core_axis_name="c", subcore_axis_name="s")
    fn = functools.partial(
        pl.kernel,
        mesh=mesh,
        out_type=jax.ShapeDtypeStruct((_NTOK, D), jnp.float32),
        scratch_types=[
            pltpu.VMEM((_TPW,), jnp.int32),
            pltpu.VMEM((_TPW,), jnp.int32),
            pltpu.VMEM((_TPW,), jnp.int32),
            pltpu.VMEM((_C, D), jnp.float32),
            pltpu.VMEM((_C, D), jnp.float32),
            pltpu.VMEM((_C, D), jnp.float32),
            pltpu.VMEM((_C, D), jnp.float32),
            [pltpu.SemaphoreType.DMA, pltpu.SemaphoreType.DMA],
            [pltpu.SemaphoreType.DMA, pltpu.SemaphoreType.DMA],
            [pltpu.SemaphoreType.DMA, pltpu.SemaphoreType.DMA],
        ],
        compiler_params=pltpu.CompilerParams(needs_layout_passes=False),
    )(_gather_add_body)
    return fn(tok, t1, t2)


def kernel(token_ids, emb_s1, emb_s2, W_fuse, b_fuse):
    w1 = W_fuse[:D]
    w2 = W_fuse[D:]
    b2d = b_fuse.reshape(1, D)
    t1, t2 = _transform_tables(emb_s1, emb_s2, w1, w2, b2d)
    tok = token_ids.reshape(-1)
    out = _gather_add(tok, t1, t2)
    return out.reshape(token_ids.shape + (D,))


# speedup vs baseline: 1.2833x; 1.0408x over previous
import functools
import math

import jax
import jax.numpy as jnp
from jax import lax
from jax.experimental import pallas as pl
from jax.experimental.pallas import tpu as pltpu
from jax.experimental.pallas import tpu_sc as plsc

D = 256
VOCAB = 8192
SCALE = math.sqrt(D)
S2_BITS = 13
S2_MASK = (1 << S2_BITS) - 1


_ROWS_PER_BLOCK = 1024
_N_BLOCKS = VOCAB // _ROWS_PER_BLOCK


def _transform_body(e1_ref, e2_ref, w1_ref, w2_ref, b_ref, t1_ref, t2_ref):
    t1_ref[...] = (
        jnp.dot(e1_ref[...], w1_ref[...], preferred_element_type=jnp.float32)
        * SCALE
        + b_ref[...]
    )
    t2_ref[...] = (
        jnp.dot(e2_ref[...], w2_ref[...], preferred_element_type=jnp.float32)
        * SCALE
    )


def _transform_tables(emb_s1, emb_s2, w1, w2, b2d):
    blk = _ROWS_PER_BLOCK
    return pl.pallas_call(
        _transform_body,
        grid=(_N_BLOCKS,),
        in_specs=[
            pl.BlockSpec((blk, D), lambda i: (i, 0)),
            pl.BlockSpec((blk, D), lambda i: (i, 0)),
            pl.BlockSpec((D, D), lambda i: (0, 0)),
            pl.BlockSpec((D, D), lambda i: (0, 0)),
            pl.BlockSpec((1, D), lambda i: (0, 0)),
        ],
        out_specs=[
            pl.BlockSpec((blk, D), lambda i: (i, 0)),
            pl.BlockSpec((blk, D), lambda i: (i, 0)),
        ],
        out_shape=[
            jax.ShapeDtypeStruct((VOCAB, D), jnp.float32),
            jax.ShapeDtypeStruct((VOCAB, D), jnp.float32),
        ],
    )(emb_s1, emb_s2, w1, w2, b2d)


_NTOK = 32768
_NW = 32
_TPW = _NTOK // _NW
_C = 64
_NCHUNK = _TPW // _C
_L = 16


def _gather_add_body(tok_hbm, t1_hbm, t2_hbm, out_hbm,
                     tok_v, idx1_v, idx2_v,
                     bufa0_v, bufa1_v, bufb0_v, bufb1_v,
                     sem_g, sem_a, sem_o):
    wid = lax.axis_index("s") * 2 + lax.axis_index("c")
    base = wid * _TPW
    bufa = [bufa0_v, bufa1_v]
    bufb = [bufb0_v, bufb1_v]

    pltpu.sync_copy(tok_hbm.at[pl.ds(base, _TPW)], tok_v)

    def split_body(j, _):
        sl = pl.ds(j * _L, _L)
        t = tok_v[sl]
        idx1_v[sl] = lax.shift_right_logical(t, S2_BITS)
        idx2_v[sl] = lax.bitwise_and(t, S2_MASK)
        return 0

    lax.fori_loop(0, _TPW // _L, split_body, 0)

    def fire_g(c, p):
        i1 = idx1_v.at[pl.ds(c * _C, _C)]
        i2 = idx2_v.at[pl.ds(c * _C, _C)]
        return (pltpu.async_copy(t1_hbm.at[i1], bufa[p], sem_g[p]),
                pltpu.async_copy(t2_hbm.at[i2], bufb[p], sem_a[p]))

    def add_chunk(p):
        a = bufa[p]
        b = bufb[p]

        def add_row(r, _):
            for k in range(D // _L):
                sl = pl.ds(k * _L, _L)
                plsc.addupdate(a.at[r, sl], b[r, sl])
            return 0

        lax.fori_loop(0, _C, add_row, 0)

    pending_o = [None, None]
    g = [None, None]
    g[0] = fire_g(0, 0)

    for c in range(_NCHUNK):
        p = c & 1
        q = 1 - p
        g[p][0].wait()
        g[p][1].wait()
        if c + 1 < _NCHUNK:
            if pending_o[q] is not None:
                pending_o[q].wait()
            g[q] = fire_g(c + 1, q)
        add_chunk(p)
        pending_o[p] = pltpu.async_copy(
            bufa[p], out_hbm.at[pl.ds(base + c * _C, _C)], sem_o[p])

    pending_o[0].wait()
    pending_o[1].wait()


def _gather_add(tok, t1, t2):
    mesh = plsc.VectorSubcoreMesh(core_axis_name="c", subcore_axis_name="s")
    fn = functools.partial(
        pl.kernel,
        mesh=mesh,
        out_type=jax.ShapeDtypeStruct((_NTOK, D), jnp.float32),
        scratch_types=[
            pltpu.VMEM((_TPW,), jnp.int32),
            pltpu.VMEM((_TPW,), jnp.int32),
            pltpu.VMEM((_TPW,), jnp.int32),
            pltpu.VMEM((_C, D), jnp.float32),
            pltpu.VMEM((_C, D), jnp.float32),
            pltpu.VMEM((_C, D), jnp.float32),
            pltpu.VMEM((_C, D), jnp.float32),
            [pltpu.SemaphoreType.DMA, pltpu.SemaphoreType.DMA],
            [pltpu.SemaphoreType.DMA, pltpu.SemaphoreType.DMA],
            [pltpu.SemaphoreType.DMA, pltpu.SemaphoreType.DMA],
        ],
        compiler_params=pltpu.CompilerParams(needs_layout_passes=False),
    )(_gather_add_body)
    return fn(tok, t1, t2)


def kernel(token_ids, emb_s1, emb_s2, W_fuse, b_fuse):
    w1 = W_fuse[:D]
    w2 = W_fuse[D:]
    b2d = b_fuse.reshape(1, D)
    t1, t2 = _transform_tables(emb_s1, emb_s2, w1, w2, b2d)
    tok = token_ids.reshape(-1)
    out = _gather_add(tok, t1, t2)
    return out.reshape(token_ids.shape + (D,))
